# baseline (speedup 1.0000x reference)
import functools

import jax
import jax.numpy as jnp
from jax import lax
from jax.experimental import pallas as pl
from jax.experimental.pallas import tpu as pltpu
from jax.experimental.pallas import tpu_sc as plsc

N_FEATURES = 64
HASHMAP_SIZE = 524288
RESOLUTION = 128.0
PRIMES3 = (73856093, 19349663, 83492791)

NUM_CORES = 2
NUM_SUBCORES = 16
NUM_WORKERS = NUM_CORES * NUM_SUBCORES
LANES = 16

CHUNK = 64
NV = N_FEATURES // LANES


def _sc_fused(xt, W, n):
    pts_per_w = n // NUM_WORKERS
    n_chunks = pts_per_w // CHUNK

    mesh = plsc.VectorSubcoreMesh(core_axis_name="c", subcore_axis_name="s")

    @functools.partial(
        pl.kernel,
        mesh=mesh,
        out_type=jax.ShapeDtypeStruct((n // 2, 2 * N_FEATURES), jnp.float32),
        scratch_types=[
            pltpu.VMEM((3, pts_per_w), jnp.float32),
            pltpu.VMEM((8, CHUNK), jnp.int32),
            pltpu.VMEM((8, CHUNK), jnp.int32),
            pltpu.VMEM((8, CHUNK), jnp.float32),
            pltpu.VMEM((8, CHUNK), jnp.float32),
            pltpu.VMEM((8, CHUNK, N_FEATURES), jnp.float32),
            pltpu.VMEM((8, CHUNK, N_FEATURES), jnp.float32),
            pltpu.VMEM((CHUNK // 2, 2 * N_FEATURES), jnp.float32),
            pltpu.VMEM((CHUNK // 2, 2 * N_FEATURES), jnp.float32),
            pltpu.SemaphoreType.DMA,
            pltpu.SemaphoreType.DMA,
            pltpu.SemaphoreType.DMA,
            pltpu.SemaphoreType.DMA,
        ],
        compiler_params=pltpu.CompilerParams(
            needs_layout_passes=False, use_tc_tiling_on_sc=False
        ),
    )
    def sc_kernel(x_hbm, w_hbm, out_hbm, xv, idx0, idx1, wgt0, wgt1,
                  rows0, rows1, outv0, outv1, semg0, semg1, semo0, semo1):
        cid = lax.axis_index("c")
        sid = lax.axis_index("s")
        wid = sid * jnp.int32(NUM_CORES) + cid
        pt0 = wid * jnp.int32(pts_per_w)

        for d in range(3):
            pltpu.sync_copy(
                x_hbm.at[pl.ds(jnp.int32(d * n) + pt0, pts_per_w)],
                xv.at[jnp.int32(d)],
            )

        idxs = (idx0, idx1)
        wgts = (wgt0, wgt1)
        rows = (rows0, rows1)
        outs = (outv0, outv1)
        semgs = (semg0, semg1)
        semos = (semo0, semo1)

        def stage(slot, g):
            idxv, wgtv, rowsv, semg = idxs[slot], wgts[slot], rows[slot], semgs[slot]
            col0 = g * jnp.int32(CHUNK)

            def grp(s, c):
                base = col0 + s * jnp.int32(LANES)
                rel = s * jnp.int32(LANES)
                a = []
                b = []
                u = []
                v = []
                for d in range(3):
                    xd = xv[jnp.int32(d), pl.ds(base, LANES)]
                    xx = (xd + 1.0) / 2.0 * RESOLUTION
                    xi = xx.astype(jnp.int32)
                    xf = xx - xi.astype(jnp.float32)
                    ad = xi * jnp.int32(PRIMES3[d])
                    a.append(ad)
                    b.append(ad + jnp.int32(PRIMES3[d]))
                    u.append(1.0 - xf)
                    v.append(xf)
                for k in range(8):
                    t0 = b[0] if k & 1 else a[0]
                    t1 = b[1] if k & 2 else a[1]
                    t2 = b[2] if k & 4 else a[2]
                    idxv[jnp.int32(k), pl.ds(rel, LANES)] = (
                        (t0 ^ t1 ^ t2) & jnp.int32(HASHMAP_SIZE - 1)
                    )
                    w0 = v[0] if k & 1 else u[0]
                    w1 = v[1] if k & 2 else u[1]
                    w2 = v[2] if k & 4 else u[2]
                    wgtv[jnp.int32(k), pl.ds(rel, LANES)] = w0 * w1 * w2
                return c

            lax.fori_loop(jnp.int32(0), jnp.int32(CHUNK // LANES), grp,
                          jnp.int32(0))
            for k in range(8):
                pltpu.async_copy(
                    w_hbm.at[idxv.at[jnp.int32(k)]],
                    rowsv.at[jnp.int32(k)],
                    semg,
                )

        def compute(slot, g, drain_out):
            idxv, wgtv, rowsv = idxs[slot], wgts[slot], rows[slot]
            outv, semg, semo = outs[slot], semgs[slot], semos[slot]
            rb = (pt0 + g * jnp.int32(CHUNK)) // jnp.int32(2)
            for k in range(8):
                pltpu.make_async_copy(
                    w_hbm.at[idxv.at[jnp.int32(k)]],
                    rowsv.at[jnp.int32(k)],
                    semg,
                ).wait()

            @pl.when(drain_out)
            def _():
                pltpu.make_async_copy(
                    outv, out_hbm.at[pl.ds(rb, CHUNK // 2)], semo
                ).wait()

            def grp(s, c):
                for i in range(LANES):
                    p = s * jnp.int32(LANES) + jnp.int32(i)
                    pv = jnp.full((LANES,), p, jnp.int32)
                    wbc = [
                        plsc.load_gather(
                            wgtv,
                            [jnp.full((LANES,), jnp.int32(k), jnp.int32), pv],
                        )
                        for k in range(8)
                    ]
                    prow = lax.shift_right_logical(p, jnp.int32(1))
                    pcol = (p & jnp.int32(1)) * jnp.int32(N_FEATURES)
                    for fv in range(NV):
                        sl = pl.ds(fv * LANES, LANES)
                        acc = wbc[0] * rowsv[jnp.int32(0), p, sl]
                        for k in range(1, 8):
                            acc = acc + wbc[k] * rowsv[jnp.int32(k), p, sl]
                        outv[prow, pl.ds(pcol + jnp.int32(fv * LANES), LANES)] = acc
                return c

            lax.fori_loop(jnp.int32(0), jnp.int32(CHUNK // LANES), grp,
                          jnp.int32(0))
            pltpu.async_copy(outv, out_hbm.at[pl.ds(rb, CHUNK // 2)], semo)

        stage(0, jnp.int32(0))

        def body(t, c):
            g0 = t * jnp.int32(2)
            stage(1, g0 + jnp.int32(1))
            compute(0, g0, g0 >= jnp.int32(2))

            @pl.when(g0 + jnp.int32(2) < jnp.int32(n_chunks))
            def _():
                stage(0, g0 + jnp.int32(2))

            compute(1, g0 + jnp.int32(1), g0 >= jnp.int32(1))
            return c

        lax.fori_loop(jnp.int32(0), jnp.int32(n_chunks // 2), body,
                      jnp.int32(0))

        last0 = (pt0 + jnp.int32((n_chunks - 2) * CHUNK)) // jnp.int32(2)
        last1 = (pt0 + jnp.int32((n_chunks - 1) * CHUNK)) // jnp.int32(2)
        pltpu.make_async_copy(
            outv0, out_hbm.at[pl.ds(last0, CHUNK // 2)], semo0
        ).wait()
        pltpu.make_async_copy(
            outv1, out_hbm.at[pl.ds(last1, CHUNK // 2)], semo1
        ).wait()

    return sc_kernel(xt, W)


def kernel(x, W):
    n = x.shape[0]
    xt = x.T.reshape(3 * n)
    out2 = _sc_fused(xt, W, n)
    return out2.reshape(n, N_FEATURES)


# speedup vs baseline: 1.0768x; 1.0768x over previous
import functools

import jax
import jax.numpy as jnp
from jax import lax
from jax.experimental import pallas as pl
from jax.experimental.pallas import tpu as pltpu
from jax.experimental.pallas import tpu_sc as plsc

N_FEATURES = 64
HASHMAP_SIZE = 524288
RESOLUTION = 128.0
PRIMES3 = (73856093, 19349663, 83492791)

NUM_CORES = 2
NUM_SUBCORES = 16
NUM_WORKERS = NUM_CORES * NUM_SUBCORES
LANES = 16

CHUNK = 64
NV = N_FEATURES // LANES


def _sc_fused(xt, W, n):
    pts_per_w = n // NUM_WORKERS
    n_chunks = pts_per_w // CHUNK

    mesh = plsc.VectorSubcoreMesh(core_axis_name="c", subcore_axis_name="s")

    @functools.partial(
        pl.kernel,
        mesh=mesh,
        out_type=jax.ShapeDtypeStruct((n, N_FEATURES), jnp.float32),
        scratch_types=[
            pltpu.VMEM((3, pts_per_w), jnp.float32),
            pltpu.VMEM((8, CHUNK), jnp.int32),
            pltpu.VMEM((8, CHUNK), jnp.int32),
            pltpu.VMEM((8, CHUNK), jnp.float32),
            pltpu.VMEM((8, CHUNK), jnp.float32),
            pltpu.VMEM((8, CHUNK, N_FEATURES), jnp.float32),
            pltpu.VMEM((8, CHUNK, N_FEATURES), jnp.float32),
            pltpu.VMEM((CHUNK, N_FEATURES), jnp.float32),
            pltpu.VMEM((CHUNK, N_FEATURES), jnp.float32),
            pltpu.SemaphoreType.DMA,
            pltpu.SemaphoreType.DMA,
            pltpu.SemaphoreType.DMA,
            pltpu.SemaphoreType.DMA,
        ],
        compiler_params=pltpu.CompilerParams(
            needs_layout_passes=False, use_tc_tiling_on_sc=False
        ),
    )
    def sc_kernel(x_hbm, w_hbm, out_hbm, xv, idx0, idx1, wgt0, wgt1,
                  rows0, rows1, outv0, outv1, semg0, semg1, semo0, semo1):
        cid = lax.axis_index("c")
        sid = lax.axis_index("s")
        wid = sid * jnp.int32(NUM_CORES) + cid
        pt0 = wid * jnp.int32(pts_per_w)

        for d in range(3):
            pltpu.sync_copy(
                x_hbm.at[pl.ds(jnp.int32(d * n) + pt0, pts_per_w)],
                xv.at[jnp.int32(d)],
            )

        idxs = (idx0, idx1)
        wgts = (wgt0, wgt1)
        rows = (rows0, rows1)
        outs = (outv0, outv1)
        semgs = (semg0, semg1)
        semos = (semo0, semo1)

        def stage(slot, g):
            idxv, wgtv, rowsv, semg = idxs[slot], wgts[slot], rows[slot], semgs[slot]
            col0 = g * jnp.int32(CHUNK)

            def grp(s, c):
                base = col0 + s * jnp.int32(LANES)
                rel = s * jnp.int32(LANES)
                a = []
                b = []
                u = []
                v = []
                for d in range(3):
                    xd = xv[jnp.int32(d), pl.ds(base, LANES)]
                    xx = (xd + 1.0) / 2.0 * RESOLUTION
                    xi = xx.astype(jnp.int32)
                    xf = xx - xi.astype(jnp.float32)
                    ad = xi * jnp.int32(PRIMES3[d])
                    a.append(ad)
                    b.append(ad + jnp.int32(PRIMES3[d]))
                    u.append(1.0 - xf)
                    v.append(xf)
                for k in range(8):
                    t0 = b[0] if k & 1 else a[0]
                    t1 = b[1] if k & 2 else a[1]
                    t2 = b[2] if k & 4 else a[2]
                    idxv[jnp.int32(k), pl.ds(rel, LANES)] = (
                        (t0 ^ t1 ^ t2) & jnp.int32(HASHMAP_SIZE - 1)
                    )
                    w0 = v[0] if k & 1 else u[0]
                    w1 = v[1] if k & 2 else u[1]
                    w2 = v[2] if k & 4 else u[2]
                    wgtv[jnp.int32(k), pl.ds(rel, LANES)] = w0 * w1 * w2
                return c

            lax.fori_loop(jnp.int32(0), jnp.int32(CHUNK // LANES), grp,
                          jnp.int32(0))
            for k in range(8):
                pltpu.async_copy(
                    w_hbm.at[idxv.at[jnp.int32(k)]],
                    rowsv.at[jnp.int32(k)],
                    semg,
                )

        def compute(slot, g, drain_out):
            idxv, wgtv, rowsv = idxs[slot], wgts[slot], rows[slot]
            outv, semg, semo = outs[slot], semgs[slot], semos[slot]
            ptb = pt0 + g * jnp.int32(CHUNK)
            for k in range(8):
                pltpu.make_async_copy(
                    w_hbm.at[idxv.at[jnp.int32(k)]],
                    rowsv.at[jnp.int32(k)],
                    semg,
                ).wait()

            @pl.when(drain_out)
            def _():
                pltpu.make_async_copy(
                    outv, out_hbm.at[pl.ds(ptb, CHUNK)], semo
                ).wait()

            def grp(s, c):
                rel = s * jnp.int32(LANES)
                wk = [
                    wgtv[jnp.int32(k), pl.ds(rel, LANES)] for k in range(8)
                ]
                dnums = lax.GatherDimensionNumbers(
                    offset_dims=(), collapsed_slice_dims=(0,),
                    start_index_map=(0,),
                )
                for i in range(LANES):
                    p = rel + jnp.int32(i)
                    iv = jnp.full((LANES, 1), jnp.int32(i), jnp.int32)
                    wbc = [
                        lax.gather(
                            wk[k], iv, dnums, (1,),
                            mode=lax.GatherScatterMode.PROMISE_IN_BOUNDS,
                        )
                        for k in range(8)
                    ]
                    for fv in range(NV):
                        sl = pl.ds(fv * LANES, LANES)
                        acc = wbc[0] * rowsv[jnp.int32(0), p, sl]
                        for k in range(1, 8):
                            acc = acc + wbc[k] * rowsv[jnp.int32(k), p, sl]
                        outv[p, sl] = acc
                return c

            lax.fori_loop(jnp.int32(0), jnp.int32(CHUNK // LANES), grp,
                          jnp.int32(0))
            pltpu.async_copy(outv, out_hbm.at[pl.ds(ptb, CHUNK)], semo)

        stage(0, jnp.int32(0))

        def body(t, c):
            g0 = t * jnp.int32(2)
            stage(1, g0 + jnp.int32(1))
            compute(0, g0, g0 >= jnp.int32(2))

            @pl.when(g0 + jnp.int32(2) < jnp.int32(n_chunks))
            def _():
                stage(0, g0 + jnp.int32(2))

            compute(1, g0 + jnp.int32(1), g0 >= jnp.int32(1))
            return c

        lax.fori_loop(jnp.int32(0), jnp.int32(n_chunks // 2), body,
                      jnp.int32(0))

        last0 = pt0 + jnp.int32((n_chunks - 2) * CHUNK)
        last1 = pt0 + jnp.int32((n_chunks - 1) * CHUNK)
        pltpu.make_async_copy(
            outv0, out_hbm.at[pl.ds(last0, CHUNK)], semo0
        ).wait()
        pltpu.make_async_copy(
            outv1, out_hbm.at[pl.ds(last1, CHUNK)], semo1
        ).wait()

    return sc_kernel(xt, W)


def kernel(x, W):
    n = x.shape[0]
    xt = x.T.reshape(3 * n)
    return _sc_fused(xt, W, n)


# speedup vs baseline: 1.0893x; 1.0116x over previous
import functools

import jax
import jax.numpy as jnp
from jax import lax
from jax.experimental import pallas as pl
from jax.experimental.pallas import tpu as pltpu
from jax.experimental.pallas import tpu_sc as plsc

N_FEATURES = 64
HASHMAP_SIZE = 524288
RESOLUTION = 128.0
PRIMES3 = (73856093, 19349663, 83492791)

NUM_CORES = 2
NUM_SUBCORES = 16
NUM_WORKERS = NUM_CORES * NUM_SUBCORES
LANES = 16

CHUNK = 64
NV = N_FEATURES // LANES


def _sc_fused(xt, W, n):
    pts_per_w = n // NUM_WORKERS
    n_chunks = pts_per_w // CHUNK

    mesh = plsc.VectorSubcoreMesh(core_axis_name="c", subcore_axis_name="s")

    @functools.partial(
        pl.kernel,
        mesh=mesh,
        out_type=jax.ShapeDtypeStruct((n, N_FEATURES), jnp.float32),
        scratch_types=[
            pltpu.VMEM((3, pts_per_w), jnp.float32),
            pltpu.VMEM((8, CHUNK), jnp.int32),
            pltpu.VMEM((8, CHUNK), jnp.int32),
            pltpu.VMEM((8, CHUNK), jnp.float32),
            pltpu.VMEM((8, CHUNK), jnp.float32),
            pltpu.VMEM((8, CHUNK, N_FEATURES), jnp.float32),
            pltpu.VMEM((8, CHUNK, N_FEATURES), jnp.float32),
            pltpu.VMEM((CHUNK, N_FEATURES), jnp.float32),
            pltpu.VMEM((CHUNK, N_FEATURES), jnp.float32),
            pltpu.SemaphoreType.DMA,
            pltpu.SemaphoreType.DMA,
            pltpu.SemaphoreType.DMA,
            pltpu.SemaphoreType.DMA,
        ],
        compiler_params=pltpu.CompilerParams(
            needs_layout_passes=False, use_tc_tiling_on_sc=False
        ),
    )
    def sc_kernel(x_hbm, w_hbm, out_hbm, xv, idx0, idx1, wgt0, wgt1,
                  rows0, rows1, outv0, outv1, semg0, semg1, semo0, semo1):
        cid = lax.axis_index("c")
        sid = lax.axis_index("s")
        wid = sid * jnp.int32(NUM_CORES) + cid
        pt0 = wid * jnp.int32(pts_per_w)

        for d in range(3):
            pltpu.sync_copy(
                x_hbm.at[pl.ds(jnp.int32(d * n) + pt0, pts_per_w)],
                xv.at[jnp.int32(d)],
            )

        idxs = (idx0, idx1)
        wgts = (wgt0, wgt1)
        rows = (rows0, rows1)
        outs = (outv0, outv1)
        semgs = (semg0, semg1)
        semos = (semo0, semo1)

        def stage(slot, g):
            idxv, wgtv, rowsv, semg = idxs[slot], wgts[slot], rows[slot], semgs[slot]
            col0 = g * jnp.int32(CHUNK)

            def grp(s, c):
                base = col0 + s * jnp.int32(LANES)
                rel = s * jnp.int32(LANES)
                a = []
                b = []
                u = []
                v = []
                for d in range(3):
                    xd = xv[jnp.int32(d), pl.ds(base, LANES)]
                    xx = (xd + 1.0) / 2.0 * RESOLUTION
                    xi = xx.astype(jnp.int32)
                    xf = xx - xi.astype(jnp.float32)
                    ad = xi * jnp.int32(PRIMES3[d])
                    a.append(ad)
                    b.append(ad + jnp.int32(PRIMES3[d]))
                    u.append(1.0 - xf)
                    v.append(xf)
                for k in range(8):
                    t0 = b[0] if k & 1 else a[0]
                    t1 = b[1] if k & 2 else a[1]
                    t2 = b[2] if k & 4 else a[2]
                    idxv[jnp.int32(k), pl.ds(rel, LANES)] = (
                        (t0 ^ t1 ^ t2) & jnp.int32(HASHMAP_SIZE - 1)
                    )
                    w0 = v[0] if k & 1 else u[0]
                    w1 = v[1] if k & 2 else u[1]
                    w2 = v[2] if k & 4 else u[2]
                    wgtv[jnp.int32(k), pl.ds(rel, LANES)] = w0 * w1 * w2
                return c

            lax.fori_loop(jnp.int32(0), jnp.int32(CHUNK // LANES), grp,
                          jnp.int32(0))
            for k in range(8):
                pltpu.async_copy(
                    w_hbm.at[idxv.at[jnp.int32(k)]],
                    rowsv.at[jnp.int32(k)],
                    semg,
                )

        def compute(slot, g, drain_out):
            idxv, wgtv, rowsv = idxs[slot], wgts[slot], rows[slot]
            outv, semg, semo = outs[slot], semgs[slot], semos[slot]
            ptb = pt0 + g * jnp.int32(CHUNK)
            for k in range(8):
                pltpu.make_async_copy(
                    w_hbm.at[idxv.at[jnp.int32(k)]],
                    rowsv.at[jnp.int32(k)],
                    semg,
                ).wait()

            @pl.when(drain_out)
            def _():
                pltpu.make_async_copy(
                    outv, out_hbm.at[pl.ds(ptb, CHUNK)], semo
                ).wait()

            def grp(s, c):
                rel = s * jnp.int32(LANES)
                wk = [
                    wgtv[jnp.int32(k), pl.ds(rel, LANES)] for k in range(8)
                ]
                dnums = lax.GatherDimensionNumbers(
                    offset_dims=(), collapsed_slice_dims=(0,),
                    start_index_map=(0,),
                )
                for i in range(LANES):
                    p = rel + jnp.int32(i)
                    iv = jnp.full((LANES, 1), jnp.int32(i), jnp.int32)
                    wbc = [
                        lax.gather(
                            wk[k], iv, dnums, (1,),
                            mode=lax.GatherScatterMode.PROMISE_IN_BOUNDS,
                        )
                        for k in range(8)
                    ]
                    for fv in range(NV):
                        sl = pl.ds(fv * LANES, LANES)
                        t = [
                            wbc[k] * rowsv[jnp.int32(k), p, sl]
                            for k in range(8)
                        ]
                        outv[p, sl] = (
                            (t[0] + t[1]) + (t[2] + t[3])
                        ) + ((t[4] + t[5]) + (t[6] + t[7]))
                return c

            lax.fori_loop(jnp.int32(0), jnp.int32(CHUNK // LANES), grp,
                          jnp.int32(0))
            pltpu.async_copy(outv, out_hbm.at[pl.ds(ptb, CHUNK)], semo)

        stage(0, jnp.int32(0))

        def body(t, c):
            g0 = t * jnp.int32(2)
            stage(1, g0 + jnp.int32(1))
            compute(0, g0, g0 >= jnp.int32(2))

            @pl.when(g0 + jnp.int32(2) < jnp.int32(n_chunks))
            def _():
                stage(0, g0 + jnp.int32(2))

            compute(1, g0 + jnp.int32(1), g0 >= jnp.int32(1))
            return c

        lax.fori_loop(jnp.int32(0), jnp.int32(n_chunks // 2), body,
                      jnp.int32(0))

        last0 = pt0 + jnp.int32((n_chunks - 2) * CHUNK)
        last1 = pt0 + jnp.int32((n_chunks - 1) * CHUNK)
        pltpu.make_async_copy(
            outv0, out_hbm.at[pl.ds(last0, CHUNK)], semo0
        ).wait()
        pltpu.make_async_copy(
            outv1, out_hbm.at[pl.ds(last1, CHUNK)], semo1
        ).wait()

    return sc_kernel(xt, W)


def kernel(x, W):
    n = x.shape[0]
    xt = x.T.reshape(3 * n)
    return _sc_fused(xt, W, n)


# speedup vs baseline: 1.0924x; 1.0029x over previous
import functools

import jax
import jax.numpy as jnp
from jax import lax
from jax.experimental import pallas as pl
from jax.experimental.pallas import tpu as pltpu
from jax.experimental.pallas import tpu_sc as plsc

N_FEATURES = 64
HASHMAP_SIZE = 524288
RESOLUTION = 128.0
PRIMES3 = (73856093, 19349663, 83492791)

NUM_CORES = 2
NUM_SUBCORES = 16
NUM_WORKERS = NUM_CORES * NUM_SUBCORES
LANES = 16

CHUNK = 64
NV = N_FEATURES // LANES


def _sc_fused(xt, W, n):
    pts_per_w = n // NUM_WORKERS
    n_chunks = pts_per_w // CHUNK

    mesh = plsc.VectorSubcoreMesh(core_axis_name="c", subcore_axis_name="s")

    @functools.partial(
        pl.kernel,
        mesh=mesh,
        out_type=jax.ShapeDtypeStruct((n, N_FEATURES), jnp.float32),
        scratch_types=[
            pltpu.VMEM((3, pts_per_w), jnp.float32),
            pltpu.VMEM((8, CHUNK), jnp.int32),
            pltpu.VMEM((8, CHUNK), jnp.int32),
            pltpu.VMEM((8, CHUNK), jnp.float32),
            pltpu.VMEM((8, CHUNK), jnp.float32),
            pltpu.VMEM((8, CHUNK, N_FEATURES), jnp.float32),
            pltpu.VMEM((8, CHUNK, N_FEATURES), jnp.float32),
            pltpu.VMEM((CHUNK, N_FEATURES), jnp.float32),
            pltpu.VMEM((CHUNK, N_FEATURES), jnp.float32),
            pltpu.SemaphoreType.DMA,
            pltpu.SemaphoreType.DMA,
            pltpu.SemaphoreType.DMA,
            pltpu.SemaphoreType.DMA,
        ],
        compiler_params=pltpu.CompilerParams(
            needs_layout_passes=False, use_tc_tiling_on_sc=False
        ),
    )
    def sc_kernel(x_hbm, w_hbm, out_hbm, xv, idx0, idx1, wgt0, wgt1,
                  rows0, rows1, outv0, outv1, semg0, semg1, semo0, semo1):
        cid = lax.axis_index("c")
        sid = lax.axis_index("s")
        wid = sid * jnp.int32(NUM_CORES) + cid
        pt0 = wid * jnp.int32(pts_per_w)

        for d in range(3):
            pltpu.sync_copy(
                x_hbm.at[pl.ds(jnp.int32(d * n) + pt0, pts_per_w)],
                xv.at[jnp.int32(d)],
            )

        idxs = (idx0, idx1)
        wgts = (wgt0, wgt1)
        rows = (rows0, rows1)
        outs = (outv0, outv1)
        semgs = (semg0, semg1)
        semos = (semo0, semo1)

        def stage(slot, g):
            idxv, wgtv, rowsv, semg = idxs[slot], wgts[slot], rows[slot], semgs[slot]
            col0 = g * jnp.int32(CHUNK)

            def grp(s, c):
                base = col0 + s * jnp.int32(LANES)
                rel = s * jnp.int32(LANES)
                a = []
                b = []
                u = []
                v = []
                for d in range(3):
                    xd = xv[jnp.int32(d), pl.ds(base, LANES)]
                    xx = (xd + 1.0) / 2.0 * RESOLUTION
                    xi = xx.astype(jnp.int32)
                    xf = xx - xi.astype(jnp.float32)
                    ad = xi * jnp.int32(PRIMES3[d])
                    a.append(ad)
                    b.append(ad + jnp.int32(PRIMES3[d]))
                    u.append(1.0 - xf)
                    v.append(xf)
                for k in range(8):
                    t0 = b[0] if k & 1 else a[0]
                    t1 = b[1] if k & 2 else a[1]
                    t2 = b[2] if k & 4 else a[2]
                    idxv[jnp.int32(k), pl.ds(rel, LANES)] = (
                        (t0 ^ t1 ^ t2) & jnp.int32(HASHMAP_SIZE - 1)
                    )
                    w0 = v[0] if k & 1 else u[0]
                    w1 = v[1] if k & 2 else u[1]
                    w2 = v[2] if k & 4 else u[2]
                    wgtv[jnp.int32(k), pl.ds(rel, LANES)] = w0 * w1 * w2
                return c

            lax.fori_loop(jnp.int32(0), jnp.int32(CHUNK // LANES), grp,
                          jnp.int32(0))
            for k in range(8):
                pltpu.async_copy(
                    w_hbm.at[idxv.at[jnp.int32(k)]],
                    rowsv.at[jnp.int32(k)],
                    semg,
                )

        def compute(slot, g, drain_out):
            idxv, wgtv, rowsv = idxs[slot], wgts[slot], rows[slot]
            outv, semg, semo = outs[slot], semgs[slot], semos[slot]
            ptb = pt0 + g * jnp.int32(CHUNK)
            for k in range(8):
                pltpu.make_async_copy(
                    w_hbm.at[idxv.at[jnp.int32(k)]],
                    rowsv.at[jnp.int32(k)],
                    semg,
                ).wait()

            @pl.when(drain_out)
            def _():
                pltpu.make_async_copy(
                    outv, out_hbm.at[pl.ds(ptb, CHUNK)], semo
                ).wait()

            @plsc.parallel_loop(jnp.int32(0), jnp.int32(CHUNK // LANES),
                                jnp.int32(1))
            def grp(s):
                rel = s * jnp.int32(LANES)
                wk = [
                    wgtv[jnp.int32(k), pl.ds(rel, LANES)] for k in range(8)
                ]
                dnums = lax.GatherDimensionNumbers(
                    offset_dims=(), collapsed_slice_dims=(0,),
                    start_index_map=(0,),
                )
                for i in range(LANES):
                    p = rel + jnp.int32(i)
                    iv = jnp.full((LANES, 1), jnp.int32(i), jnp.int32)
                    wbc = [
                        lax.gather(
                            wk[k], iv, dnums, (1,),
                            mode=lax.GatherScatterMode.PROMISE_IN_BOUNDS,
                        )
                        for k in range(8)
                    ]
                    for fv in range(NV):
                        sl = pl.ds(fv * LANES, LANES)
                        t = [
                            wbc[k] * rowsv[jnp.int32(k), p, sl]
                            for k in range(8)
                        ]
                        outv[p, sl] = (
                            (t[0] + t[1]) + (t[2] + t[3])
                        ) + ((t[4] + t[5]) + (t[6] + t[7]))

            pltpu.async_copy(outv, out_hbm.at[pl.ds(ptb, CHUNK)], semo)

        stage(0, jnp.int32(0))

        def body(t, c):
            g0 = t * jnp.int32(2)
            stage(1, g0 + jnp.int32(1))
            compute(0, g0, g0 >= jnp.int32(2))

            @pl.when(g0 + jnp.int32(2) < jnp.int32(n_chunks))
            def _():
                stage(0, g0 + jnp.int32(2))

            compute(1, g0 + jnp.int32(1), g0 >= jnp.int32(1))
            return c

        lax.fori_loop(jnp.int32(0), jnp.int32(n_chunks // 2), body,
                      jnp.int32(0))

        last0 = pt0 + jnp.int32((n_chunks - 2) * CHUNK)
        last1 = pt0 + jnp.int32((n_chunks - 1) * CHUNK)
        pltpu.make_async_copy(
            outv0, out_hbm.at[pl.ds(last0, CHUNK)], semo0
        ).wait()
        pltpu.make_async_copy(
            outv1, out_hbm.at[pl.ds(last1, CHUNK)], semo1
        ).wait()

    return sc_kernel(xt, W)


def kernel(x, W):
    n = x.shape[0]
    xt = x.T.reshape(3 * n)
    return _sc_fused(xt, W, n)
